# Initial kernel scaffold; baseline (speedup 1.0000x reference)
import jax
import jax.numpy as jnp
from jax.experimental import pallas as pl


def kernel(predicted_locs, predicted_scores, boxes, labels, priors_cxcy):
    raise NotImplementedError("write your pallas kernel here")


# speedup vs baseline: 13.3167x; 13.3167x over previous
import jax
import jax.numpy as jnp
from jax.experimental import pallas as pl
from jax.experimental.pallas import tpu as pltpu

_B = 32
_P = 8732
_NOBJ = 16
_NCLS = 81
_ROWS = 72
_LANES = 128
_PPAD = _ROWS * _LANES
_THRESH = 0.5
_NEG_RATIO = 3.0


def _rmax(x):
    return jnp.max(jnp.max(x, axis=1, keepdims=True), axis=0, keepdims=True)


def _rmin(x):
    return jnp.min(jnp.min(x, axis=1, keepdims=True), axis=0, keepdims=True)


def _rsum(x):
    return jnp.sum(jnp.sum(x, axis=1, keepdims=True), axis=0, keepdims=True)


def _mbl_kernel(boxes_ref, labels_ref, pcxcy_ref, plocs_ref, scores_ref,
                out_ref):
    ridx = jax.lax.broadcasted_iota(jnp.int32, (_ROWS, _LANES), 0)
    cidx = jax.lax.broadcasted_iota(jnp.int32, (_ROWS, _LANES), 1)
    pidx = (ridx * _LANES + cidx).astype(jnp.float32)
    valid = pidx < float(_P)

    pcx = pcxcy_ref[0]
    pcy = pcxcy_ref[1]
    pw = pcxcy_ref[2]
    ph = pcxcy_ref[3]
    px0 = pcx - pw * 0.5
    py0 = pcy - ph * 0.5
    px1 = pcx + pw * 0.5
    py1 = pcy + ph * 0.5
    parea = pw * ph

    zero = jnp.zeros((_ROWS, _LANES), jnp.float32)
    ove_m = jnp.full((_ROWS, _LANES), -1.0, jnp.float32)
    lbl_m = zero
    bx0_m = zero
    by0_m = zero
    bx1_m = zero
    by1_m = zero
    forced = zero
    lbl_f = zero
    bx0_f = zero
    by0_f = zero
    bx1_f = zero
    by1_f = zero

    for o in range(_NOBJ):
        bx0 = boxes_ref[0, o, 0]
        by0 = boxes_ref[0, o, 1]
        bx1 = boxes_ref[0, o, 2]
        by1 = boxes_ref[0, o, 3]
        lbl_o = labels_ref[0, 0, o].astype(jnp.float32)
        barea = (bx1 - bx0) * (by1 - by0)
        iw = jnp.maximum(jnp.minimum(px1, bx1) - jnp.maximum(px0, bx0), 0.0)
        ih = jnp.maximum(jnp.minimum(py1, by1) - jnp.maximum(py0, by0), 0.0)
        inter = iw * ih
        iou = inter / (parea + barea - inter)
        row = jnp.where(valid, iou, -1.0)
        upd = row > ove_m
        ove_m = jnp.where(upd, row, ove_m)
        lbl_m = jnp.where(upd, lbl_o, lbl_m)
        bx0_m = jnp.where(upd, bx0, bx0_m)
        by0_m = jnp.where(upd, by0, by0_m)
        bx1_m = jnp.where(upd, bx1, bx1_m)
        by1_m = jnp.where(upd, by1, by1_m)
        rmax = _rmax(row)
        p_o = _rmin(jnp.where(row == rmax, pidx, 1e9))
        fmask = pidx == p_o
        forced = jnp.where(fmask, 1.0, forced)
        lbl_f = jnp.where(fmask, lbl_o, lbl_f)
        bx0_f = jnp.where(fmask, bx0, bx0_f)
        by0_f = jnp.where(fmask, by0, by0_f)
        bx1_f = jnp.where(fmask, bx1, bx1_f)
        by1_f = jnp.where(fmask, by1, by1_f)

    isf = forced > 0.0
    ove = jnp.where(isf, 1.0, ove_m)
    lbl = jnp.where(isf, lbl_f, lbl_m)
    gx0 = jnp.where(isf, bx0_f, bx0_m)
    gy0 = jnp.where(isf, by0_f, by0_m)
    gx1 = jnp.where(isf, bx1_f, bx1_m)
    gy1 = jnp.where(isf, by1_f, by1_m)
    lbl = jnp.where(ove < _THRESH, 0.0, lbl)
    positive = lbl > 0.0

    cx = (gx0 + gx1) * 0.5
    cy = (gy0 + gy1) * 0.5
    w = gx1 - gx0
    h = gy1 - gy0
    t0 = (cx - pcx) / (pw * 0.1)
    t1 = (cy - pcy) / (ph * 0.1)
    t2 = jnp.log(w / pw) * 5.0
    t3 = jnp.log(h / ph) * 5.0
    la = (jnp.abs(plocs_ref[0, 0] - t0) + jnp.abs(plocs_ref[0, 1] - t1) +
          jnp.abs(plocs_ref[0, 2] - t2) + jnp.abs(plocs_ref[0, 3] - t3))
    loc_sum = _rsum(jnp.where(positive, la, 0.0))
    n_posf = _rsum(jnp.where(positive, 1.0, 0.0))

    m = scores_ref[0, 0]
    for c in range(1, _NCLS):
        m = jnp.maximum(m, scores_ref[0, c])
    acc = zero
    sel = zero
    for c in range(_NCLS):
        s = scores_ref[0, c]
        acc = acc + jnp.exp(s - m)
        sel = jnp.where(lbl == float(c), s, sel)
    ce = m + jnp.log(acc) - sel
    pos_sum = _rsum(jnp.where(positive, ce, 0.0))
    vneg = jnp.where(jnp.logical_and(valid, jnp.logical_not(positive)), ce,
                     zero)
    vneg = jnp.maximum(vneg, 0.0)

    u = jax.lax.bitcast_convert_type(vneg, jnp.int32)
    k = jnp.minimum(_NEG_RATIO * n_posf, float(_P))
    prefix = jnp.zeros((1, 1), jnp.int32)
    for b in range(30, -1, -1):
        cand = prefix | jnp.int32(1 << b)
        cnt = _rsum(jnp.where(u >= cand, 1.0, 0.0))
        prefix = jnp.where(cnt >= k, cand, prefix)
    tval = jax.lax.bitcast_convert_type(prefix, jnp.float32)
    gt = u > prefix
    cnt_gt = _rsum(jnp.where(gt, 1.0, 0.0))
    sum_gt = _rsum(jnp.where(gt, vneg, 0.0))
    hard_sum = sum_gt + (k - cnt_gt) * tval

    loss = (hard_sum + pos_sum) / n_posf + loc_sum / (n_posf * 4.0)
    out_ref[0] = jnp.broadcast_to(loss, (8, _LANES))


def kernel(predicted_locs, predicted_scores, boxes, labels, priors_cxcy):
    pc = jnp.pad(jnp.transpose(priors_cxcy), ((0, 0), (0, _PPAD - _P)),
                 constant_values=1.0).reshape(4, _ROWS, _LANES)
    pl_t = jnp.pad(jnp.transpose(predicted_locs, (0, 2, 1)),
                   ((0, 0), (0, 0), (0, _PPAD - _P)))
    pl_t = pl_t.reshape(_B, 4, _ROWS, _LANES)
    sc_t = jnp.pad(jnp.transpose(predicted_scores, (0, 2, 1)),
                   ((0, 0), (0, 0), (0, _PPAD - _P)))
    sc_t = sc_t.reshape(_B, _NCLS, _ROWS, _LANES)
    labels32 = labels.astype(jnp.int32).reshape(_B, 1, _NOBJ)

    out = pl.pallas_call(
        _mbl_kernel,
        grid=(_B,),
        in_specs=[
            pl.BlockSpec((1, _NOBJ, 4), lambda b: (b, 0, 0),
                         memory_space=pltpu.SMEM),
            pl.BlockSpec((1, 1, _NOBJ), lambda b: (b, 0, 0),
                         memory_space=pltpu.SMEM),
            pl.BlockSpec((4, _ROWS, _LANES), lambda b: (0, 0, 0)),
            pl.BlockSpec((1, 4, _ROWS, _LANES), lambda b: (b, 0, 0, 0)),
            pl.BlockSpec((1, _NCLS, _ROWS, _LANES), lambda b: (b, 0, 0, 0)),
        ],
        out_specs=pl.BlockSpec((1, 8, _LANES), lambda b: (b, 0, 0)),
        out_shape=jax.ShapeDtypeStruct((_B, 8, _LANES), jnp.float32),
    )(boxes, labels32, pc, pl_t, sc_t)
    return out[:, 0, 0]


# speedup vs baseline: 16.8678x; 1.2667x over previous
import jax
import jax.numpy as jnp
from jax.experimental import pallas as pl
from jax.experimental.pallas import tpu as pltpu

_B = 32
_P = 8732
_NOBJ = 16
_NCLS = 81
_ROWS = 72
_LANES = 128
_PPAD = _ROWS * _LANES
_THRESH = 0.5
_NEG_RATIO = 3.0


def _rmax(x):
    return jnp.max(jnp.max(x, axis=1, keepdims=True), axis=0, keepdims=True)


def _rmin(x):
    return jnp.min(jnp.min(x, axis=1, keepdims=True), axis=0, keepdims=True)


def _rsum(x):
    return jnp.sum(jnp.sum(x, axis=1, keepdims=True), axis=0, keepdims=True)


def _mbl_kernel(boxes_ref, labels_ref, pcxcy_ref, plocs_ref, scores_ref,
                out_ref):
    ridx = jax.lax.broadcasted_iota(jnp.int32, (_ROWS, _LANES), 0)
    cidx = jax.lax.broadcasted_iota(jnp.int32, (_ROWS, _LANES), 1)
    pidx = (ridx * _LANES + cidx).astype(jnp.float32)
    valid = pidx < float(_P)

    pcx = pcxcy_ref[0]
    pcy = pcxcy_ref[1]
    pw = pcxcy_ref[2]
    ph = pcxcy_ref[3]
    px0 = pcx - pw * 0.5
    py0 = pcy - ph * 0.5
    px1 = pcx + pw * 0.5
    py1 = pcy + ph * 0.5
    parea = pw * ph

    zero = jnp.zeros((_ROWS, _LANES), jnp.float32)
    ove_m = jnp.full((_ROWS, _LANES), -1.0, jnp.float32)
    lbl_m = zero
    bx0_m = zero
    by0_m = zero
    bx1_m = zero
    by1_m = zero
    forced = zero
    lbl_f = zero
    bx0_f = zero
    by0_f = zero
    bx1_f = zero
    by1_f = zero

    for o in range(_NOBJ):
        bx0 = boxes_ref[0, o, 0]
        by0 = boxes_ref[0, o, 1]
        bx1 = boxes_ref[0, o, 2]
        by1 = boxes_ref[0, o, 3]
        lbl_o = labels_ref[0, 0, o].astype(jnp.float32)
        barea = (bx1 - bx0) * (by1 - by0)
        iw = jnp.maximum(jnp.minimum(px1, bx1) - jnp.maximum(px0, bx0), 0.0)
        ih = jnp.maximum(jnp.minimum(py1, by1) - jnp.maximum(py0, by0), 0.0)
        inter = iw * ih
        iou = inter / (parea + barea - inter)
        row = jnp.where(valid, iou, -1.0)
        upd = row > ove_m
        ove_m = jnp.where(upd, row, ove_m)
        lbl_m = jnp.where(upd, lbl_o, lbl_m)
        bx0_m = jnp.where(upd, bx0, bx0_m)
        by0_m = jnp.where(upd, by0, by0_m)
        bx1_m = jnp.where(upd, bx1, bx1_m)
        by1_m = jnp.where(upd, by1, by1_m)
        rmax = _rmax(row)
        p_o = _rmin(jnp.where(row == rmax, pidx, 1e9))
        fmask = pidx == p_o
        forced = jnp.where(fmask, 1.0, forced)
        lbl_f = jnp.where(fmask, lbl_o, lbl_f)
        bx0_f = jnp.where(fmask, bx0, bx0_f)
        by0_f = jnp.where(fmask, by0, by0_f)
        bx1_f = jnp.where(fmask, bx1, bx1_f)
        by1_f = jnp.where(fmask, by1, by1_f)

    isf = forced > 0.0
    ove = jnp.where(isf, 1.0, ove_m)
    lbl = jnp.where(isf, lbl_f, lbl_m)
    gx0 = jnp.where(isf, bx0_f, bx0_m)
    gy0 = jnp.where(isf, by0_f, by0_m)
    gx1 = jnp.where(isf, bx1_f, bx1_m)
    gy1 = jnp.where(isf, by1_f, by1_m)
    lbl = jnp.where(ove < _THRESH, 0.0, lbl)
    positive = lbl > 0.0

    cx = (gx0 + gx1) * 0.5
    cy = (gy0 + gy1) * 0.5
    w = gx1 - gx0
    h = gy1 - gy0
    t0 = (cx - pcx) / (pw * 0.1)
    t1 = (cy - pcy) / (ph * 0.1)
    t2 = jnp.log(w / pw) * 5.0
    t3 = jnp.log(h / ph) * 5.0
    la = (jnp.abs(plocs_ref[0, 0] - t0) + jnp.abs(plocs_ref[0, 1] - t1) +
          jnp.abs(plocs_ref[0, 2] - t2) + jnp.abs(plocs_ref[0, 3] - t3))
    loc_sum = _rsum(jnp.where(positive, la, 0.0))
    n_posf = _rsum(jnp.where(positive, 1.0, 0.0))

    ciota = jax.lax.broadcasted_iota(jnp.int32, (_NCLS, _LANES), 0)
    ce_rows = []
    for r in range(69):
        base = r * _LANES if r < 68 else _P - _LANES
        t = jnp.transpose(scores_ref[0, base:base + _LANES, :])
        mrow = jnp.max(t, axis=0, keepdims=True)
        arow = jnp.sum(jnp.exp(t - mrow), axis=0, keepdims=True)
        lrow = lbl[r:r + 1, :].astype(jnp.int32)
        if r == 68:
            lrow = pltpu.roll(lrow, 100, axis=1)
        srow = jnp.sum(jnp.where(ciota == lrow, t, 0.0), axis=0,
                       keepdims=True)
        crow = mrow + jnp.log(arow) - srow
        if r == 68:
            crow = pltpu.roll(crow, 28, axis=1)
        ce_rows.append(crow)
    ce_rows.append(jnp.zeros((3, _LANES), jnp.float32))
    ce = jnp.concatenate(ce_rows, axis=0)
    pos_sum = _rsum(jnp.where(positive, ce, 0.0))
    vneg = jnp.where(jnp.logical_and(valid, jnp.logical_not(positive)), ce,
                     zero)
    vneg = jnp.maximum(vneg, 0.0)

    u = jax.lax.bitcast_convert_type(vneg, jnp.int32)
    k = jnp.minimum(_NEG_RATIO * n_posf, float(_P))
    prefix = jnp.zeros((1, 1), jnp.int32)
    for b in range(29, -1, -2):
        c01 = prefix | jnp.int32(1 << b)
        c10 = prefix | jnp.int32(2 << b)
        c11 = prefix | jnp.int32(3 << b)
        n01 = _rsum(jnp.where(u >= c01, 1.0, 0.0))
        n10 = _rsum(jnp.where(u >= c10, 1.0, 0.0))
        n11 = _rsum(jnp.where(u >= c11, 1.0, 0.0))
        prefix = jnp.where(n11 >= k, c11,
                           jnp.where(n10 >= k, c10,
                                     jnp.where(n01 >= k, c01, prefix)))
    cand0 = prefix | jnp.int32(1)
    cnt0 = _rsum(jnp.where(u >= cand0, 1.0, 0.0))
    prefix = jnp.where(cnt0 >= k, cand0, prefix)
    tval = jax.lax.bitcast_convert_type(prefix, jnp.float32)
    gt = u > prefix
    cnt_gt = _rsum(jnp.where(gt, 1.0, 0.0))
    sum_gt = _rsum(jnp.where(gt, vneg, 0.0))
    hard_sum = sum_gt + (k - cnt_gt) * tval

    loss = (hard_sum + pos_sum) / n_posf + loc_sum / (n_posf * 4.0)
    out_ref[0] = jnp.broadcast_to(loss, (8, _LANES))


def kernel(predicted_locs, predicted_scores, boxes, labels, priors_cxcy):
    pc = jnp.pad(jnp.transpose(priors_cxcy), ((0, 0), (0, _PPAD - _P)),
                 constant_values=1.0).reshape(4, _ROWS, _LANES)
    pl_t = jnp.pad(jnp.transpose(predicted_locs, (0, 2, 1)),
                   ((0, 0), (0, 0), (0, _PPAD - _P)))
    pl_t = pl_t.reshape(_B, 4, _ROWS, _LANES)
    labels32 = labels.astype(jnp.int32).reshape(_B, 1, _NOBJ)

    out = pl.pallas_call(
        _mbl_kernel,
        grid=(_B,),
        in_specs=[
            pl.BlockSpec((1, _NOBJ, 4), lambda b: (b, 0, 0),
                         memory_space=pltpu.SMEM),
            pl.BlockSpec((1, 1, _NOBJ), lambda b: (b, 0, 0),
                         memory_space=pltpu.SMEM),
            pl.BlockSpec((4, _ROWS, _LANES), lambda b: (0, 0, 0)),
            pl.BlockSpec((1, 4, _ROWS, _LANES), lambda b: (b, 0, 0, 0)),
            pl.BlockSpec((1, _P, _NCLS), lambda b: (b, 0, 0)),
        ],
        out_specs=pl.BlockSpec((1, 8, _LANES), lambda b: (b, 0, 0)),
        out_shape=jax.ShapeDtypeStruct((_B, 8, _LANES), jnp.float32),
    )(boxes, labels32, pc, pl_t, predicted_scores)
    return out[:, 0, 0]
